# baseline (speedup 1.0000x reference)
import jax
import jax.numpy as jnp
from jax import lax
from jax.experimental import pallas as pl
from jax.experimental.pallas import tpu as pltpu
from jax.experimental.pallas import tpu_sc as plsc

CUTOFF = 5.0
KB = 10000
LANES = 16
TC_BLK = 128000


def _splat(x, dtype=jnp.float32):
    return lax.broadcast_in_dim(jnp.asarray(x, dtype), (LANES,), ())


def _num_workers():
    try:
        info = plsc.get_sparse_core_info()
        return info.num_cores, info.num_subcores
    except Exception:
        return 2, 16


def _make_splitter(e_in):
    grid = pl.cdiv(e_in, TC_BLK)

    def split_body(vt_ref, p0_ref, p1_ref, p2_ref):
        x = vt_ref[...]
        p0_ref[...] = x[0]
        p1_ref[...] = x[1]
        p2_ref[...] = x[2]

    return pl.pallas_call(
        split_body,
        grid=(grid,),
        in_specs=[pl.BlockSpec((3, TC_BLK), lambda i: (0, i))],
        out_specs=[
            pl.BlockSpec((TC_BLK,), lambda i: (i,)),
            pl.BlockSpec((TC_BLK,), lambda i: (i,)),
            pl.BlockSpec((TC_BLK,), lambda i: (i,)),
        ],
        out_shape=[jax.ShapeDtypeStruct((e_in,), jnp.float32)] * 3,
    )


def _make_merger(e_f, e_in):
    grid = pl.cdiv(e_f, TC_BLK)

    def merge_body(v0_ref, v1_ref, v2_ref, d_ref, i_ref, vt_ref, df_ref,
                   sw_ref, m_ref):
        oob = i_ref[...] >= e_in
        v = jnp.stack([v0_ref[...], v1_ref[...], v2_ref[...]], axis=0)
        vt_ref[...] = jnp.where(oob[None, :], CUTOFF, v)
        d = jnp.where(oob, CUTOFF, d_ref[...])
        df_ref[...] = d
        edge = d < CUTOFF
        sw = 0.5 * jnp.cos(jnp.pi * (d * (1.0 / CUTOFF))) + 0.5
        sw_ref[...] = jnp.where(edge, sw, 0.0)
        m_ref[...] = edge

    return pl.pallas_call(
        merge_body,
        grid=(grid,),
        in_specs=[pl.BlockSpec((TC_BLK,), lambda i: (i,))] * 5,
        out_specs=[
            pl.BlockSpec((3, TC_BLK), lambda i: (0, i)),
            pl.BlockSpec((TC_BLK,), lambda i: (i,)),
            pl.BlockSpec((TC_BLK,), lambda i: (i,)),
            pl.BlockSpec((TC_BLK,), lambda i: (i,)),
        ],
        out_shape=[
            jax.ShapeDtypeStruct((3, e_f), jnp.float32),
            jax.ShapeDtypeStruct((e_f,), jnp.float32),
            jax.ShapeDtypeStruct((e_f,), jnp.float32),
            jax.ShapeDtypeStruct((e_f,), jnp.bool_),
        ],
    )


def kernel(vec, distances, filter_indices):
    e_in = vec.shape[0]
    e_f = filter_indices.shape[0]
    nc, ns = _num_workers()
    nw = nc * ns
    assert e_f % KB == 0, e_f
    nblk = e_f // KB
    tmax = pl.cdiv(nblk, nw)
    ngrp = KB // LANES

    assert tmax % 2 == 0, tmax

    def body(p0_hbm, p1_hbm, p2_hbm, dist_hbm, fidx_hbm, vf0_hbm, vf1_hbm,
             vf2_hbm, distf_hbm, idx_b0, idx_b1, cidx_b0, cidx_b1,
             dist_b0, dist_b1, vc0_b0, vc0_b1, vc1_b0, vc1_b1, vc2_b0,
             vc2_b1, sem_g0, sem_g1, sem_w0, sem_w1):
        wid = lax.axis_index("s") * nc + lax.axis_index("c")
        e_in_v = _splat(e_in, jnp.int32)
        neg1 = _splat(-1, jnp.int32)
        idx_b = (idx_b0, idx_b1)
        cidx_b = (cidx_b0, cidx_b1)
        sem_g = (sem_g0, sem_g1)
        sem_w = (sem_w0, sem_w1)
        bufs = (
            ((dist_b0, distf_hbm), (vc0_b0, vf0_hbm), (vc1_b0, vf1_hbm),
             (vc2_b0, vf2_hbm)),
            ((dist_b1, distf_hbm), (vc0_b1, vf0_hbm), (vc1_b1, vf1_hbm),
             (vc2_b1, vf2_hbm)),
        )
        gsrc = (
            ((dist_b0, dist_hbm), (vc0_b0, p0_hbm), (vc1_b0, p1_hbm),
             (vc2_b0, p2_hbm)),
            ((dist_b1, dist_hbm), (vc0_b1, p0_hbm), (vc1_b1, p1_hbm),
             (vc2_b1, p2_hbm)),
        )

        def stage_in(t, si, drain):
            b = wid + nw * t

            @pl.when(b < nblk)
            def _():
                base = b * KB
                pltpu.sync_copy(fidx_hbm.at[pl.ds(base, KB)], idx_b[si])

                def clamp_body(j, c):
                    s16 = pl.ds(j * LANES, LANES)
                    iv = idx_b[si][s16]
                    cidx_b[si][s16] = jnp.where(iv < e_in_v, iv, neg1)
                    return c

                lax.fori_loop(0, ngrp, clamp_body, 0)
                if drain:

                    @pl.when(t >= 2)
                    def _():
                        for buf, hb in bufs[si]:
                            pltpu.make_async_copy(
                                buf, hb.at[pl.ds(base, KB)],
                                sem_w[si]).wait()

                idx = plsc.Indices(cidx_b[si], ignored_value=-1)
                for buf, hb in gsrc[si]:
                    pltpu.async_copy(hb.at[idx], buf, sem_g[si])

        def stage_out(t, so):
            b = wid + nw * t

            @pl.when(b < nblk)
            def _():
                base = b * KB
                idx = plsc.Indices(cidx_b[so], ignored_value=-1)
                for buf, hb in gsrc[so]:
                    pltpu.make_async_copy(hb.at[idx], buf,
                                          sem_g[so]).wait()

                @pl.when(b + 2 * nw < nblk)
                def _():
                    for buf, hb in bufs[so]:
                        pltpu.async_copy(buf, hb.at[pl.ds(base, KB)],
                                         sem_w[so])

                @pl.when(b + 2 * nw >= nblk)
                def _():
                    for buf, hb in bufs[so]:
                        pltpu.sync_copy(buf, hb.at[pl.ds(base, KB)])

        stage_in(jnp.int32(0), 0, drain=False)

        def g_body(g, carry):
            for tt in range(2):
                t = 2 * g + tt
                stage_in(t + 1, 1 - tt, drain=True)
                stage_out(t, tt)
            return carry

        lax.fori_loop(0, tmax // 2, g_body, 0)

    mesh = plsc.VectorSubcoreMesh(core_axis_name="c", subcore_axis_name="s")
    run = pl.kernel(
        body,
        out_type=(
            jax.ShapeDtypeStruct((e_f,), jnp.float32),
            jax.ShapeDtypeStruct((e_f,), jnp.float32),
            jax.ShapeDtypeStruct((e_f,), jnp.float32),
            jax.ShapeDtypeStruct((e_f,), jnp.float32),
        ),
        mesh=mesh,
        compiler_params=pltpu.CompilerParams(needs_layout_passes=False),
        scratch_types=(
            [pltpu.VMEM((KB,), jnp.int32)] * 4
            + [pltpu.VMEM((KB,), jnp.float32)] * 8
            + [pltpu.SemaphoreType.DMA] * 4
        ),
    )
    p0, p1, p2 = _make_splitter(e_in)(vec.T)
    vf0, vf1, vf2, dist_g = run(p0, p1, p2, distances, filter_indices)
    vecf_t, dist_f, switch, mask = _make_merger(e_f, e_in)(
        vf0, vf1, vf2, dist_g, filter_indices)
    return vecf_t.T, dist_f, switch, mask


# speedup vs baseline: 1.0267x; 1.0267x over previous
import jax
import jax.numpy as jnp
from jax import lax
from jax.experimental import pallas as pl
from jax.experimental.pallas import tpu as pltpu
from jax.experimental.pallas import tpu_sc as plsc

CUTOFF = 5.0
KB = 10000
LANES = 16
TC_BLK = 128000


def _splat(x, dtype=jnp.float32):
    return lax.broadcast_in_dim(jnp.asarray(x, dtype), (LANES,), ())


def _num_workers():
    try:
        info = plsc.get_sparse_core_info()
        return info.num_cores, info.num_subcores
    except Exception:
        return 2, 16


def _make_splitter(e_in):
    grid = pl.cdiv(e_in, TC_BLK)

    def split_body(vt_ref, p0_ref, p1_ref, p2_ref):
        x = vt_ref[...]
        p0_ref[...] = x[0]
        p1_ref[...] = x[1]
        p2_ref[...] = x[2]

    return pl.pallas_call(
        split_body,
        grid=(grid,),
        in_specs=[pl.BlockSpec((3, TC_BLK), lambda i: (0, i))],
        out_specs=[
            pl.BlockSpec((TC_BLK,), lambda i: (i,)),
            pl.BlockSpec((TC_BLK,), lambda i: (i,)),
            pl.BlockSpec((TC_BLK,), lambda i: (i,)),
        ],
        out_shape=[jax.ShapeDtypeStruct((e_in,), jnp.float32)] * 3,
    )


def _make_merger(e_f, e_in):
    grid = pl.cdiv(e_f, TC_BLK)

    def merge_body(v0_ref, v1_ref, v2_ref, d_ref, i_ref, vt_ref, df_ref,
                   sw_ref, m_ref):
        oob = i_ref[...] >= e_in
        v = jnp.stack([v0_ref[...], v1_ref[...], v2_ref[...]], axis=0)
        vt_ref[...] = jnp.where(oob[None, :], CUTOFF, v)
        d = jnp.where(oob, CUTOFF, d_ref[...])
        df_ref[...] = d
        edge = d < CUTOFF
        sw = 0.5 * jnp.cos(jnp.pi * (d * (1.0 / CUTOFF))) + 0.5
        sw_ref[...] = jnp.where(edge, sw, 0.0)
        m_ref[...] = edge

    return pl.pallas_call(
        merge_body,
        grid=(grid,),
        in_specs=[pl.BlockSpec((TC_BLK,), lambda i: (i,))] * 5,
        out_specs=[
            pl.BlockSpec((3, TC_BLK), lambda i: (0, i)),
            pl.BlockSpec((TC_BLK,), lambda i: (i,)),
            pl.BlockSpec((TC_BLK,), lambda i: (i,)),
            pl.BlockSpec((TC_BLK,), lambda i: (i,)),
        ],
        out_shape=[
            jax.ShapeDtypeStruct((3, e_f), jnp.float32),
            jax.ShapeDtypeStruct((e_f,), jnp.float32),
            jax.ShapeDtypeStruct((e_f,), jnp.float32),
            jax.ShapeDtypeStruct((e_f,), jnp.bool_),
        ],
    )


def _make_sc_gather(e_in, e_f, nst):
    nc, ns = _num_workers()
    nw = nc * ns
    assert e_f % KB == 0, e_f
    nblk = e_f // KB
    tmax = pl.cdiv(nblk, nw)
    ngrp = KB // LANES
    assert tmax % 2 == 0, tmax

    def body(*args):
        srcs = args[:nst]
        fidx_hbm = args[nst]
        outs = args[nst + 1:2 * nst + 1]
        scr = args[2 * nst + 1:]
        idx_b = scr[0:2]
        cidx_b = scr[2:4]
        data = scr[4:4 + 2 * nst]
        sem_g = scr[4 + 2 * nst:6 + 2 * nst]
        sem_w = scr[6 + 2 * nst:8 + 2 * nst]
        wid = lax.axis_index("s") * nc + lax.axis_index("c")
        e_in_v = _splat(e_in, jnp.int32)
        neg1 = _splat(-1, jnp.int32)
        bufs = tuple(
            tuple((data[s * nst + k], outs[k]) for k in range(nst))
            for s in (0, 1))
        gsrc = tuple(
            tuple((data[s * nst + k], srcs[k]) for k in range(nst))
            for s in (0, 1))

        def stage_in(t, si, drain):
            b = wid + nw * t

            @pl.when(b < nblk)
            def _():
                base = b * KB
                pltpu.sync_copy(fidx_hbm.at[pl.ds(base, KB)], idx_b[si])

                def clamp_body(j, c):
                    s16 = pl.ds(j * LANES, LANES)
                    iv = idx_b[si][s16]
                    cidx_b[si][s16] = jnp.where(iv < e_in_v, iv, neg1)
                    return c

                lax.fori_loop(0, ngrp, clamp_body, 0)
                if drain:

                    @pl.when(t >= 2)
                    def _():
                        for buf, hb in bufs[si]:
                            pltpu.make_async_copy(
                                buf, hb.at[pl.ds(base, KB)],
                                sem_w[si]).wait()

                idx = plsc.Indices(cidx_b[si], ignored_value=-1)
                for buf, hb in gsrc[si]:
                    pltpu.async_copy(hb.at[idx], buf, sem_g[si])

        def stage_out(t, so):
            b = wid + nw * t

            @pl.when(b < nblk)
            def _():
                base = b * KB
                idx = plsc.Indices(cidx_b[so], ignored_value=-1)
                for buf, hb in gsrc[so]:
                    pltpu.make_async_copy(hb.at[idx], buf,
                                          sem_g[so]).wait()

                @pl.when(b + 2 * nw < nblk)
                def _():
                    for buf, hb in bufs[so]:
                        pltpu.async_copy(buf, hb.at[pl.ds(base, KB)],
                                         sem_w[so])

                @pl.when(b + 2 * nw >= nblk)
                def _():
                    for buf, hb in bufs[so]:
                        pltpu.sync_copy(buf, hb.at[pl.ds(base, KB)])

        stage_in(jnp.int32(0), 0, drain=False)

        def g_body(g, carry):
            for tt in range(2):
                t = 2 * g + tt
                stage_in(t + 1, 1 - tt, drain=True)
                stage_out(t, tt)
            return carry

        lax.fori_loop(0, tmax // 2, g_body, 0)

    mesh = plsc.VectorSubcoreMesh(core_axis_name="c", subcore_axis_name="s")
    return pl.kernel(
        body,
        out_type=(jax.ShapeDtypeStruct((e_f,), jnp.float32),) * nst,
        mesh=mesh,
        compiler_params=pltpu.CompilerParams(needs_layout_passes=False),
        scratch_types=(
            [pltpu.VMEM((KB,), jnp.int32)] * 4
            + [pltpu.VMEM((KB,), jnp.float32)] * (2 * nst)
            + [pltpu.SemaphoreType.DMA] * 4
        ),
    )


def kernel(vec, distances, filter_indices):
    e_in = vec.shape[0]
    e_f = filter_indices.shape[0]
    (dist_g,) = _make_sc_gather(e_in, e_f, 1)(distances, filter_indices)
    p0, p1, p2 = _make_splitter(e_in)(vec.T)
    vf0, vf1, vf2 = _make_sc_gather(e_in, e_f, 3)(p0, p1, p2,
                                                  filter_indices)
    vecf_t, dist_f, switch, mask = _make_merger(e_f, e_in)(
        vf0, vf1, vf2, dist_g, filter_indices)
    return vecf_t.T, dist_f, switch, mask
